# baseline (speedup 1.0000x reference)
import functools

import jax
import jax.numpy as jnp
import numpy as np
from jax.experimental import pallas as pl

_KK = 5000
_EPS = 1e-5
_PPAD = 5120
_BLOCK = 512


def _shift_sel(hin, hout):
    S = np.zeros((3, hin, hout), np.float32)
    for d in range(3):
        for yo in range(hout):
            S[d, yo + d, yo] = 1.0
    return jnp.asarray(S)


def _conv_mat(w, hin, hout):
    O, C = w.shape[0], w.shape[1]
    S = _shift_sel(hin, hout)
    M = jnp.einsum('ocde,dyz,exw->cyxozw', w, S, S)
    return M.reshape(C * hin * hin, O * hout * hout)


def _conv_mat_up(w, hout):
    O, C = w.shape[0], w.shape[1]
    Q = np.zeros((3, 4, hout), np.float32)
    for d in range(3):
        for yo in range(hout):
            Q[d, (yo + d) // 2, yo] = 1.0
    Q = jnp.asarray(Q)
    M = jnp.einsum('ocde,dyz,exw->cyxozw', w, Q, Q)
    return M.reshape(C * 4 * 4, O * hout * hout)


def _stage_kernel(x_ref, w_ref, a_ref, c_ref, y_ref, s_ref, *, relu):
    i = pl.program_id(0)
    x = x_ref[...] * a_ref[0, :][None, :] + c_ref[0, :][None, :]
    if relu:
        x = jnp.maximum(x, 0.0)
    rid = jax.lax.broadcasted_iota(jnp.int32, x.shape, 0) + i * _BLOCK
    x = jnp.where(rid < _KK, x, 0.0)
    y = jnp.dot(x, w_ref[...], preferred_element_type=jnp.float32)
    y_ref[...] = y

    @pl.when(i == 0)
    def _():
        s_ref[...] = jnp.zeros_like(s_ref)

    s_ref[0:1, :] += jnp.sum(y, axis=0, keepdims=True)
    s_ref[1:2, :] += jnp.sum(y * y, axis=0, keepdims=True)


def _stage(x, W, a, c, relu):
    P, Cin = x.shape
    Ncols = W.shape[1]
    grid = P // _BLOCK
    y, s = pl.pallas_call(
        functools.partial(_stage_kernel, relu=relu),
        grid=(grid,),
        in_specs=[
            pl.BlockSpec((_BLOCK, Cin), lambda i: (i, 0)),
            pl.BlockSpec((Cin, Ncols), lambda i: (0, 0)),
            pl.BlockSpec((1, Cin), lambda i: (0, 0)),
            pl.BlockSpec((1, Cin), lambda i: (0, 0)),
        ],
        out_specs=[
            pl.BlockSpec((_BLOCK, Ncols), lambda i: (i, 0)),
            pl.BlockSpec((8, Ncols), lambda i: (0, 0)),
        ],
        out_shape=[
            jax.ShapeDtypeStruct((P, Ncols), jnp.float32),
            jax.ShapeDtypeStruct((8, Ncols), jnp.float32),
        ],
    )(x, W, a, c)
    return y, s


def _affine(s, g, b, O, S_sp):
    s2 = s[0:2].reshape(2, O, S_sp).sum(-1)
    count = _KK * S_sp
    m = s2[0] / count
    v = s2[1] / count - m * m
    a = g * jax.lax.rsqrt(v + _EPS)
    c = b - m * a
    return jnp.repeat(a, S_sp)[None, :], jnp.repeat(c, S_sp)[None, :]


def kernel(pha, err, hid, org_shape, w1, g1, b1, w2, g2, b2, w3, g3, b3, w4, c4b):
    B, _, Hq, Wq = err.shape
    H, W = 4 * Hq, 4 * Wq
    kk = _KK

    ef = err.reshape(B, -1)
    _, idx = jax.lax.top_k(ef, kk)
    ref = jnp.zeros_like(ef).at[jnp.arange(B)[:, None], idx].set(1.0)
    ref = (ref * (ef > 0).astype(jnp.float32)).reshape(B, 1, Hq, Wq)

    flat = idx.reshape(-1)
    ih = flat // Wq
    iw = flat % Wq
    ib = jnp.zeros((kk,), flat.dtype)

    x = jnp.concatenate([hid, pha], axis=1)
    t = jnp.arange(5)
    rows_q = jnp.clip(ih[:, None] - 2 + t[None, :], 0, Hq - 1)
    cols_q = jnp.clip(iw[:, None] - 2 + t[None, :], 0, Wq - 1)
    qwin = x[0][:, rows_q[:, :, None], cols_q[:, None, :]]

    Rb = np.zeros((8, 5), np.float32)
    for j in range(8):
        if j % 2 == 0:
            k0, f = (j - 3 + 1) // 2 + 1, 0.25
        else:
            k0, f = (j - 3) // 2 + 1, 0.75
        Rb[j, k0] = 1.0 - f
        Rb[j, k0 + 1] = f
    Rb = jnp.asarray(Rb)
    j8 = jnp.arange(8)
    rmask = ((2 * ih[:, None] - 3 + j8[None, :] >= 0)
             & (2 * ih[:, None] - 3 + j8[None, :] <= 2 * Hq - 1)).astype(jnp.float32)
    cmask = ((2 * iw[:, None] - 3 + j8[None, :] >= 0)
             & (2 * iw[:, None] - 3 + j8[None, :] <= 2 * Wq - 1)).astype(jnp.float32)
    patches = jnp.einsum('jy,cpyx,kx->cpjk', Rb, qwin, Rb)
    patches = patches * rmask[None, :, :, None] * cmask[None, :, None, :]
    patches = patches.transpose(1, 0, 2, 3)

    xflat = jnp.pad(patches.reshape(kk, 33 * 64), ((0, _PPAD - kk), (0, 0)))

    W1 = _conv_mat(w1, 8, 6)
    W2 = _conv_mat(w2, 6, 4)
    W3 = _conv_mat_up(w3, 6)
    W4 = _conv_mat(w4, 6, 4)
    ones = jnp.ones((1, 33 * 64), jnp.float32)
    zeros = jnp.zeros((1, 33 * 64), jnp.float32)

    y1, s1 = _stage(xflat, W1, ones, zeros, relu=False)
    a1, c1 = _affine(s1, g1, b1, 24, 36)
    y2, s2 = _stage(y1, W2, a1, c1, relu=True)
    a2, c2 = _affine(s2, g2, b2, 16, 16)
    y3, s3 = _stage(y2, W3, a2, c2, relu=True)
    a3, c3 = _affine(s3, g3, b3, 12, 36)
    y4, _ = _stage(y3, W4, a3, c3, relu=True)
    y = (y4[:kk] + c4b[0]).reshape(kk, 4, 4)

    pha_full = jax.image.resize(pha, (B, 1, H, W), method='bilinear')
    dn = jax.lax.ScatterDimensionNumbers(
        update_window_dims=(1, 2), inserted_window_dims=(),
        scatter_dims_to_operand_dims=(0, 1))
    img = jax.lax.scatter(
        pha_full[0, 0], jnp.stack([ih * 4, iw * 4], axis=-1), y, dn,
        indices_are_sorted=False, unique_indices=True)
    pha_out = img[None, None]
    return (pha_out, ref)


# speedup vs baseline: 7.9110x; 7.9110x over previous
import functools

import jax
import jax.numpy as jnp
import numpy as np
from jax.experimental import pallas as pl

_KK = 5000
_EPS = 1e-5
_PPAD = 5120
_BLOCK = 512


def _shift_sel(hin, hout):
    S = np.zeros((3, hin, hout), np.float32)
    for d in range(3):
        for yo in range(hout):
            S[d, yo + d, yo] = 1.0
    return jnp.asarray(S)


def _conv_mat(w, hin, hout):
    O, C = w.shape[0], w.shape[1]
    S = _shift_sel(hin, hout)
    M = jnp.einsum('ocde,dyz,exw->cyxozw', w, S, S)
    return M.reshape(C * hin * hin, O * hout * hout)


def _conv_mat_up(w, hout):
    O, C = w.shape[0], w.shape[1]
    Q = np.zeros((3, 4, hout), np.float32)
    for d in range(3):
        for yo in range(hout):
            Q[d, (yo + d) // 2, yo] = 1.0
    Q = jnp.asarray(Q)
    M = jnp.einsum('ocde,dyz,exw->cyxozw', w, Q, Q)
    return M.reshape(C * 4 * 4, O * hout * hout)


def _stage_kernel(x_ref, w_ref, a_ref, c_ref, y_ref, s_ref, *, relu):
    i = pl.program_id(0)
    x = x_ref[...] * a_ref[0, :][None, :] + c_ref[0, :][None, :]
    if relu:
        x = jnp.maximum(x, 0.0)
    rid = jax.lax.broadcasted_iota(jnp.int32, x.shape, 0) + i * _BLOCK
    x = jnp.where(rid < _KK, x, 0.0)
    y = jnp.dot(x, w_ref[...], preferred_element_type=jnp.float32)
    y_ref[...] = y

    @pl.when(i == 0)
    def _():
        s_ref[...] = jnp.zeros_like(s_ref)

    s_ref[0:1, :] += jnp.sum(y, axis=0, keepdims=True)
    s_ref[1:2, :] += jnp.sum(y * y, axis=0, keepdims=True)


def _stage(x, W, a, c, relu):
    P, Cin = x.shape
    Ncols = W.shape[1]
    grid = P // _BLOCK
    y, s = pl.pallas_call(
        functools.partial(_stage_kernel, relu=relu),
        grid=(grid,),
        in_specs=[
            pl.BlockSpec((_BLOCK, Cin), lambda i: (i, 0)),
            pl.BlockSpec((Cin, Ncols), lambda i: (0, 0)),
            pl.BlockSpec((1, Cin), lambda i: (0, 0)),
            pl.BlockSpec((1, Cin), lambda i: (0, 0)),
        ],
        out_specs=[
            pl.BlockSpec((_BLOCK, Ncols), lambda i: (i, 0)),
            pl.BlockSpec((8, Ncols), lambda i: (0, 0)),
        ],
        out_shape=[
            jax.ShapeDtypeStruct((P, Ncols), jnp.float32),
            jax.ShapeDtypeStruct((8, Ncols), jnp.float32),
        ],
    )(x, W, a, c)
    return y, s


def _affine(s, g, b, O, S_sp):
    s2 = s[0:2].reshape(2, O, S_sp).sum(-1)
    count = _KK * S_sp
    m = s2[0] / count
    v = s2[1] / count - m * m
    a = g * jax.lax.rsqrt(v + _EPS)
    c = b - m * a
    return jnp.repeat(a, S_sp)[None, :], jnp.repeat(c, S_sp)[None, :]


def kernel(pha, err, hid, org_shape, w1, g1, b1, w2, g2, b2, w3, g3, b3, w4, c4b):
    B, _, Hq, Wq = err.shape
    H, W = 4 * Hq, 4 * Wq
    kk = _KK

    ef = err.reshape(B, -1)
    _, idx = jax.lax.top_k(ef, kk)
    ref = jnp.zeros_like(ef).at[jnp.arange(B)[:, None], idx].set(1.0)
    ref = (ref * (ef > 0).astype(jnp.float32)).reshape(B, 1, Hq, Wq)

    flat = idx.reshape(-1)
    ih = flat // Wq
    iw = flat % Wq
    ib = jnp.zeros((kk,), flat.dtype)

    x = jnp.concatenate([hid, pha], axis=1)
    xh = jax.image.resize(x, (B, 33, 2 * Hq, 2 * Wq), method='bilinear')
    j8 = jnp.arange(8)
    r_half = (ih * 2 - 3)[:, None] + j8[None, :]
    c_half = (iw * 2 - 3)[:, None] + j8[None, :]
    rows = jnp.clip(r_half, 0, 2 * Hq - 1)[:, None, :, None]
    cols = jnp.clip(c_half, 0, 2 * Wq - 1)[:, None, None, :]
    patches = xh[ib[:, None, None, None], jnp.arange(33)[None, :, None, None],
                 rows, cols]
    rmask = ((r_half >= 0) & (r_half <= 2 * Hq - 1)).astype(jnp.float32)
    cmask = ((c_half >= 0) & (c_half <= 2 * Wq - 1)).astype(jnp.float32)
    patches = patches * rmask[:, None, :, None] * cmask[:, None, None, :]

    xflat = jnp.pad(patches.reshape(kk, 33 * 64), ((0, _PPAD - kk), (0, 0)))

    W1 = _conv_mat(w1, 8, 6)
    W2 = _conv_mat(w2, 6, 4)
    W3 = _conv_mat_up(w3, 6)
    W4 = _conv_mat(w4, 6, 4)
    ones = jnp.ones((1, 33 * 64), jnp.float32)
    zeros = jnp.zeros((1, 33 * 64), jnp.float32)

    y1, s1 = _stage(xflat, W1, ones, zeros, relu=False)
    a1, c1 = _affine(s1, g1, b1, 24, 36)
    y2, s2 = _stage(y1, W2, a1, c1, relu=True)
    a2, c2 = _affine(s2, g2, b2, 16, 16)
    y3, s3 = _stage(y2, W3, a2, c2, relu=True)
    a3, c3 = _affine(s3, g3, b3, 12, 36)
    y4, _ = _stage(y3, W4, a3, c3, relu=True)
    y = (y4[:kk] + c4b[0]).reshape(kk, 1, 4, 4)

    pha_full = jax.image.resize(pha, (B, 1, H, W), method='bilinear')
    p = pha_full.reshape(B, 1, H // 4, 4, W // 4, 4).transpose(0, 2, 4, 1, 3, 5)
    p = p.at[ib, ih, iw].set(y)
    pha_out = p.transpose(0, 3, 1, 4, 2, 5).reshape(B, 1, H, W)
    return (pha_out, ref)


# speedup vs baseline: 9.8258x; 1.2420x over previous
import functools

import jax
import jax.numpy as jnp
import numpy as np
from jax.experimental import pallas as pl

_KK = 5000
_EPS = 1e-5
_PPAD = 5120
_BLOCK = 512


def _shift_sel(hin, hout):
    S = np.zeros((3, hin, hout), np.float32)
    for d in range(3):
        for yo in range(hout):
            S[d, yo + d, yo] = 1.0
    return jnp.asarray(S)


def _conv_mat(w, hin, hout):
    O, C = w.shape[0], w.shape[1]
    S = _shift_sel(hin, hout)
    M = jnp.einsum('ocde,dyz,exw->cyxozw', w, S, S)
    return M.reshape(C * hin * hin, O * hout * hout)


def _conv_mat_up(w, hout):
    O, C = w.shape[0], w.shape[1]
    Q = np.zeros((3, 4, hout), np.float32)
    for d in range(3):
        for yo in range(hout):
            Q[d, (yo + d) // 2, yo] = 1.0
    Q = jnp.asarray(Q)
    M = jnp.einsum('ocde,dyz,exw->cyxozw', w, Q, Q)
    return M.reshape(C * 4 * 4, O * hout * hout)


def _stage_kernel(x_ref, w_ref, a_ref, c_ref, y_ref, s_ref, *, relu):
    i = pl.program_id(0)
    x = x_ref[...] * a_ref[0, :][None, :] + c_ref[0, :][None, :]
    if relu:
        x = jnp.maximum(x, 0.0)
    rid = jax.lax.broadcasted_iota(jnp.int32, x.shape, 0) + i * _BLOCK
    x = jnp.where(rid < _KK, x, 0.0)
    y = jnp.dot(x, w_ref[...], preferred_element_type=jnp.float32)
    y_ref[...] = y

    @pl.when(i == 0)
    def _():
        s_ref[...] = jnp.zeros_like(s_ref)

    s_ref[0:1, :] += jnp.sum(y, axis=0, keepdims=True)
    s_ref[1:2, :] += jnp.sum(y * y, axis=0, keepdims=True)


def _stage(x, W, a, c, relu):
    P, Cin = x.shape
    Ncols = W.shape[1]
    grid = P // _BLOCK
    y, s = pl.pallas_call(
        functools.partial(_stage_kernel, relu=relu),
        grid=(grid,),
        in_specs=[
            pl.BlockSpec((_BLOCK, Cin), lambda i: (i, 0)),
            pl.BlockSpec((Cin, Ncols), lambda i: (0, 0)),
            pl.BlockSpec((1, Cin), lambda i: (0, 0)),
            pl.BlockSpec((1, Cin), lambda i: (0, 0)),
        ],
        out_specs=[
            pl.BlockSpec((_BLOCK, Ncols), lambda i: (i, 0)),
            pl.BlockSpec((8, Ncols), lambda i: (0, 0)),
        ],
        out_shape=[
            jax.ShapeDtypeStruct((P, Ncols), jnp.float32),
            jax.ShapeDtypeStruct((8, Ncols), jnp.float32),
        ],
    )(x, W, a, c)
    return y, s


def _affine(s, g, b, O, S_sp):
    s2 = s[0:2].reshape(2, O, S_sp).sum(-1)
    count = _KK * S_sp
    m = s2[0] / count
    v = s2[1] / count - m * m
    a = g * jax.lax.rsqrt(v + _EPS)
    c = b - m * a
    return jnp.repeat(a, S_sp)[None, :], jnp.repeat(c, S_sp)[None, :]


def kernel(pha, err, hid, org_shape, w1, g1, b1, w2, g2, b2, w3, g3, b3, w4, c4b):
    B, _, Hq, Wq = err.shape
    H, W = 4 * Hq, 4 * Wq
    kk = _KK

    ef = err.reshape(B, -1)
    _, idx = jax.lax.top_k(ef, kk)
    ref = jnp.zeros_like(ef).at[jnp.arange(B)[:, None], idx].set(1.0)
    ref = (ref * (ef > 0).astype(jnp.float32)).reshape(B, 1, Hq, Wq)

    flat = idx.reshape(-1)
    ih = flat // Wq
    iw = flat % Wq
    ib = jnp.zeros((kk,), flat.dtype)

    x = jnp.concatenate([hid, pha], axis=1)
    xh = jax.image.resize(x, (B, 33, 2 * Hq, 2 * Wq), method='bilinear')
    xp = jnp.pad(xh, ((0, 0), (0, 0), (3, 3), (3, 3)))
    rows = (ih * 2)[:, None, None, None] + jnp.arange(8)[None, None, :, None]
    cols = (iw * 2)[:, None, None, None] + jnp.arange(8)[None, None, None, :]
    patches = xp[ib[:, None, None, None], jnp.arange(33)[None, :, None, None],
                 rows, cols]

    xflat = jnp.pad(patches.reshape(kk, 33 * 64), ((0, _PPAD - kk), (0, 0)))

    W1 = _conv_mat(w1, 8, 6)
    W2 = _conv_mat(w2, 6, 4)
    W3 = _conv_mat_up(w3, 6)
    W4 = _conv_mat(w4, 6, 4)
    ones = jnp.ones((1, 33 * 64), jnp.float32)
    zeros = jnp.zeros((1, 33 * 64), jnp.float32)

    y1, s1 = _stage(xflat, W1, ones, zeros, relu=False)
    a1, c1 = _affine(s1, g1, b1, 24, 36)
    y2, s2 = _stage(y1, W2, a1, c1, relu=True)
    a2, c2 = _affine(s2, g2, b2, 16, 16)
    y3, s3 = _stage(y2, W3, a2, c2, relu=True)
    a3, c3 = _affine(s3, g3, b3, 12, 36)
    y4, _ = _stage(y3, W4, a3, c3, relu=True)
    y = (y4[:kk] + c4b[0]).reshape(kk, 1, 4, 4)

    pha_full = jax.image.resize(pha, (B, 1, H, W), method='bilinear')
    p = pha_full.reshape(B, 1, H // 4, 4, W // 4, 4).transpose(0, 2, 4, 1, 3, 5)
    p = p.at[ib, ih, iw].set(y)
    pha_out = p.transpose(0, 3, 1, 4, 2, 5).reshape(B, 1, H, W)
    return (pha_out, ref)
